# baseline (speedup 1.0000x reference)
import functools
import math

import jax
import jax.numpy as jnp
from jax import lax
from jax.experimental import pallas as pl
from jax.experimental.pallas import tpu as pltpu
from jax.experimental.pallas import tpu_sc as plsc

_VOCAB = 100000
_DIM = 128
_B = 4096
_L = 50
_NC = 2
_NS = 16
_NW = _NC * _NS
_RPW = _B // _NW
_LANES = 16
_SCALE = math.sqrt(_DIM)
_RING = 5


def _build_sc_kernel():
    mesh = plsc.VectorSubcoreMesh(core_axis_name="c", subcore_axis_name="s")

    @functools.partial(
        pl.kernel,
        mesh=mesh,
        out_type=jax.ShapeDtypeStruct((_L, _B, _DIM), jnp.float32),
        scratch_types=[
            pltpu.VMEM((_L, _RPW), jnp.int32),
            pltpu.VMEM((_RING, _RPW, _DIM), jnp.float32),
            pltpu.SemaphoreType.DMA,
            pltpu.SemaphoreType.DMA,
        ],
    )
    def k(table_hbm, xt_hbm, out_hbm, idx_v, rows_v, gsem, osem):
        wid = lax.axis_index("s") * _NC + lax.axis_index("c")
        base = wid * _RPW
        pltpu.sync_copy(xt_hbm.at[:, pl.ds(base, _RPW)], idx_v)
        for p in range(3):
            pltpu.async_copy(table_hbm.at[idx_v.at[p]], rows_v.at[p], gsem)

        def outer(h, _):
            for b in range(_RING):
                l = _RING * h + b
                nxt = (b + 3) % _RING

                @pl.when(l >= 2)
                def _wait_prev_write():
                    pltpu.make_async_copy(
                        rows_v.at[nxt], out_hbm.at[0, pl.ds(base, _RPW)], osem
                    ).wait()

                @pl.when(l + 3 < _L)
                def _start_next_gather():
                    pltpu.async_copy(
                        table_hbm.at[idx_v.at[l + 3]], rows_v.at[nxt], gsem
                    )

                pltpu.make_async_copy(
                    table_hbm.at[idx_v.at[l]], rows_v.at[b], gsem
                ).wait()

                for c in range(_DIM // _LANES):
                    @plsc.parallel_loop(0, _RPW, unroll=8)
                    def _scale(r):
                        sl = pl.ds(c * _LANES, _LANES)
                        rows_v[b, r, sl] = rows_v[b, r, sl] * _SCALE

                pltpu.async_copy(
                    rows_v.at[b], out_hbm.at[l, pl.ds(base, _RPW)], osem
                )
            return 0

        lax.fori_loop(0, _L // _RING, outer, 0)
        for p in range(2):
            pltpu.make_async_copy(
                rows_v.at[p], out_hbm.at[0, pl.ds(base, _RPW)], osem
            ).wait()

    return k


_sc_gather = _build_sc_kernel()


def kernel(x, table):
    out_lbd = _sc_gather(table, x.T)
    return out_lbd.transpose(1, 0, 2)


# speedup vs baseline: 1.0307x; 1.0307x over previous
import functools
import math

import jax
import jax.numpy as jnp
from jax import lax
from jax.experimental import pallas as pl
from jax.experimental.pallas import tpu as pltpu
from jax.experimental.pallas import tpu_sc as plsc

_VOCAB = 100000
_DIM = 128
_B = 4096
_L = 50
_NC = 2
_NS = 16
_NW = _NC * _NS
_RPW = _B // _NW
_LANES = 16
_SCALE = math.sqrt(_DIM)
_RING = 5


def _build_sc_kernel():
    mesh = plsc.VectorSubcoreMesh(core_axis_name="c", subcore_axis_name="s")

    @functools.partial(
        pl.kernel,
        mesh=mesh,
        out_type=jax.ShapeDtypeStruct((_L, _B, _DIM), jnp.float32),
        scratch_types=[
            pltpu.VMEM((_L, _RPW), jnp.int32),
            pltpu.VMEM((_RING, _RPW, _DIM), jnp.float32),
            pltpu.SemaphoreType.DMA,
            pltpu.SemaphoreType.DMA,
        ],
    )
    def k(table_hbm, xt_hbm, out_hbm, idx_v, rows_v, gsem, osem):
        wid = lax.axis_index("s") * _NC + lax.axis_index("c")
        base = wid * _RPW
        pltpu.sync_copy(xt_hbm.at[:, pl.ds(base, _RPW)], idx_v)
        for p in range(3):
            pltpu.async_copy(table_hbm.at[idx_v.at[p]], rows_v.at[p], gsem)

        def outer(h, _):
            for b in range(_RING):
                l = _RING * h + b
                nxt = (b + 3) % _RING

                @pl.when(l >= 2)
                def _wait_prev_write():
                    pltpu.make_async_copy(
                        rows_v.at[nxt], out_hbm.at[0, pl.ds(base, _RPW)], osem
                    ).wait()

                @pl.when(l + 3 < _L)
                def _start_next_gather():
                    pltpu.async_copy(
                        table_hbm.at[idx_v.at[l + 3]], rows_v.at[nxt], gsem
                    )

                pltpu.make_async_copy(
                    table_hbm.at[idx_v.at[l]], rows_v.at[b], gsem
                ).wait()

                @plsc.parallel_loop(0, _RPW * _DIM // _LANES, unroll=8)
                def _scale(j):
                    r = lax.shift_right_logical(j, 3)
                    sl = pl.ds(lax.shift_left(j & 7, 4), _LANES)
                    rows_v[b, r, sl] = rows_v[b, r, sl] * _SCALE

                pltpu.async_copy(
                    rows_v.at[b], out_hbm.at[l, pl.ds(base, _RPW)], osem
                )
            return 0

        lax.fori_loop(0, _L // _RING, outer, 0)
        for p in range(2):
            pltpu.make_async_copy(
                rows_v.at[p], out_hbm.at[0, pl.ds(base, _RPW)], osem
            ).wait()

    return k


_sc_gather = _build_sc_kernel()


def kernel(x, table):
    out_lbd = _sc_gather(table, x.T)
    return out_lbd.transpose(1, 0, 2)
